# baseline (speedup 1.0000x reference)
import functools

import jax
import jax.numpy as jnp
from jax import lax
from jax.experimental import pallas as pl
from jax.experimental.pallas import tpu as pltpu
from jax.experimental.pallas import tpu_sc as plsc

B = 16384
UD = 16
CD = 8
ND = 8
H1 = 64
H2 = 32

_NC = 2
_NS = 16
_NW = _NC * _NS
_BPW = B // _NW
_CHUNK = 128
_NCH = _BPW // _CHUNK


def _make_gather():
    mesh = plsc.VectorSubcoreMesh(core_axis_name="c", subcore_axis_name="s")

    @functools.partial(
        pl.kernel,
        mesh=mesh,
        out_type=[
            jax.ShapeDtypeStruct((B, UD), jnp.float32),
            jax.ShapeDtypeStruct((B, UD), jnp.float32),
        ],
        scratch_types=[
            pltpu.VMEM((_NCH, _CHUNK), jnp.int32),
            pltpu.VMEM((_NCH, _CHUNK), jnp.int32),
            pltpu.VMEM((_BPW, UD), jnp.float32),
            pltpu.VMEM((_BPW, UD), jnp.float32),
            pltpu.SemaphoreType.DMA,
        ],
        compiler_params=pltpu.CompilerParams(use_tc_tiling_on_sc=False),
    )
    def gather_k(uid_hbm, cid_hbm, utab_hbm, ctab_hbm, u_out, c_out,
                 uidx_v, cidx_v, urows_v, crows_v, sem):
        wid = lax.axis_index("s") * _NC + lax.axis_index("c")
        base = wid * _BPW
        for j in range(_NCH):
            pltpu.sync_copy(uid_hbm.at[pl.ds(base + j * _CHUNK, _CHUNK)],
                            uidx_v.at[j])
            pltpu.sync_copy(cid_hbm.at[pl.ds(base + j * _CHUNK, _CHUNK)],
                            cidx_v.at[j])
        copies = []
        for j in range(_NCH):
            copies.append(pltpu.async_copy(
                utab_hbm.at[uidx_v.at[j]],
                urows_v.at[pl.ds(j * _CHUNK, _CHUNK)], sem))
            copies.append(pltpu.async_copy(
                ctab_hbm.at[cidx_v.at[j]],
                crows_v.at[pl.ds(j * _CHUNK, _CHUNK)], sem))
        for cp in copies:
            cp.wait()
        pltpu.sync_copy(urows_v, u_out.at[pl.ds(base, _BPW)])
        pltpu.sync_copy(crows_v, c_out.at[pl.ds(base, _BPW)])

    return gather_k


_gather_cache = []


def _gather(*args):
    if not _gather_cache:
        _gather_cache.append(_make_gather())
    return _gather_cache[0](*args)


def _mlp_body(u_ref, c_ref, n_ref, w1u_ref, w1c_ref, w1n_ref, b1_ref,
              w2_ref, b2_ref, w3_ref, b3_ref, out_ref):
    h = (jnp.dot(u_ref[...], w1u_ref[...], preferred_element_type=jnp.float32)
         + jnp.dot(c_ref[...], w1c_ref[...], preferred_element_type=jnp.float32)
         + jnp.dot(n_ref[...], w1n_ref[...], preferred_element_type=jnp.float32)
         + b1_ref[...])
    h = jnp.maximum(h, 0.0)
    h = jnp.maximum(
        jnp.dot(h, w2_ref[...], preferred_element_type=jnp.float32) + b2_ref[...],
        0.0)
    out_ref[...] = (jnp.dot(h, w3_ref[...], preferred_element_type=jnp.float32)
                    + b3_ref[...])


def kernel(user_id, cat_id, numeric, user_table, cat_table, W1, b1, W2, b2, W3, b3):
    ctab16 = jnp.pad(cat_table, ((0, 0), (0, UD - CD)))
    u, c16 = _gather(user_id.astype(jnp.int32), cat_id.astype(jnp.int32),
                     user_table, ctab16)
    w1t = W1.T
    w1u = w1t[:UD]
    w1c = jnp.pad(w1t[UD:UD + CD], ((0, UD - CD), (0, 0)))
    w1n = w1t[UD + CD:]
    out = pl.pallas_call(
        _mlp_body,
        out_shape=jax.ShapeDtypeStruct((B, 1), jnp.float32),
    )(u, c16, numeric, w1u, w1c, w1n, b1.reshape(1, H1),
      W2.T, b2.reshape(1, H2), W3.T, b3.reshape(1, 1))
    return out.reshape(B)
